# Initial kernel scaffold; baseline (speedup 1.0000x reference)
import jax
import jax.numpy as jnp
from jax.experimental import pallas as pl


def kernel(x, edge_index, W1, a_s1, a_d1, b1, W2, a_s2, a_d2, b2):
    raise NotImplementedError("write your pallas kernel here")


# speedup vs baseline: 33.9248x; 33.9248x over previous
import functools

import jax
import jax.numpy as jnp
from jax import lax
from jax.experimental import pallas as pl
from jax.experimental.pallas import tpu as pltpu
from jax.experimental.pallas import tpu_sc as plsc

NC = 2
NS = 16
LN = 16

F_IN = 128
H = 8
C1 = 16
C2 = 40
SRC1_W = 144
DST1_W = 32
ACC1_W = 144
SRC2_W = 176
DST2_W = 32
ACC2_W = 176
K_E = 80
K_E2 = 32


def _blockdiag(a):
    h, c = a.shape
    eye = jnp.eye(h, dtype=a.dtype)
    return (eye[:, None, :] * a[:, :, None]).reshape(h * c, h)


def _prep1_body(x_ref, w1_ref, asf_ref, adf_ref, src_ref, asad_ref):
    n = x_ref.shape[0]
    h1 = jnp.dot(x_ref[...], w1_ref[...], preferred_element_type=jnp.float32)
    as1 = jnp.dot(h1, asf_ref[...], preferred_element_type=jnp.float32)
    ad1 = jnp.dot(h1, adf_ref[...], preferred_element_type=jnp.float32)
    z8 = jnp.zeros((n, H), jnp.float32)
    src_ref[...] = jnp.concatenate([h1, as1, z8], axis=1)
    asad_ref[...] = jnp.concatenate([as1, ad1], axis=1)


def _dst1_body(asad_ref, dst_ref):
    asad = asad_ref[...]
    as1 = asad[:, 0:H]
    ad1 = asad[:, H:2 * H]
    maxas = jnp.max(as1, axis=0, keepdims=True)
    m1 = jnp.maximum(0.0, maxas + ad1)
    n = asad.shape[0]
    z8 = jnp.zeros((n, H), jnp.float32)
    dst_ref[...] = jnp.concatenate([ad1, z8, m1, z8], axis=1)


def _edge1_body(src_t, dst_t, sidx, didx, out_hbm,
                acc, zbuf, msg, rows_s, rows_d, sv, dv, sem1, sem2):
    n = acc.shape[0]
    cid = lax.axis_index("c")
    sid = lax.axis_index("s")
    wid = cid * NS + sid
    e_total = sidx.shape[0]
    e_per_w = e_total // (NC * NS)
    zrows = zbuf.shape[0]
    nchunks = n // zrows
    per_s = (nchunks + NS - 1) // NS

    def zero_row(r, _):
        for j in range(ACC1_W // LN):
            zbuf[r, pl.ds(LN * j, LN)] = jnp.zeros((LN,), jnp.float32)
        return 0
    lax.fori_loop(0, zrows, zero_row, 0)
    for i in range(per_s):
        ci = sid + NS * i
        @pl.when(ci < nchunks)
        def _():
            pltpu.sync_copy(zbuf, acc.at[pl.ds(zrows * ci, zrows)])
    plsc.subcore_barrier()

    lane = lax.iota(jnp.int32, LN)
    ebase0 = wid * e_per_w

    def chunk_body(t, _):
        base = ebase0 + t * K_E
        pltpu.sync_copy(sidx.at[pl.ds(base, K_E)], sv)
        pltpu.sync_copy(didx.at[pl.ds(base, K_E)], dv)
        cp1 = pltpu.async_copy(src_t.at[sv], rows_s, sem1)
        cp2 = pltpu.async_copy(dst_t.at[dv], rows_d, sem2)
        cp1.wait()
        cp2.wait()

        def edge_body(k, _):
            asv = rows_s[k, pl.ds(F_IN, LN)]
            adv = rows_d[k, pl.ds(0, LN)]
            mv = rows_d[k, pl.ds(LN, LN)]
            tv = asv + adv
            ev = jnp.maximum(tv, 0.2 * tv)
            w = jnp.exp(ev - mv)
            w = jnp.where(lane < H, w, 0.0)
            for hh in range(H):
                wh = _lane_bcast(w, hh)
                msg[k, pl.ds(LN * hh, LN)] = rows_s[k, pl.ds(LN * hh, LN)] * wh
            msg[k, pl.ds(F_IN, LN)] = w
            return 0
        lax.fori_loop(0, K_E, edge_body, 0)
        pltpu.sync_copy(msg, acc.at[dv], add=True)
        return 0
    lax.fori_loop(0, e_per_w // K_E, chunk_body, 0)
    plsc.subcore_barrier()

    for i in range(per_s):
        ci = sid + NS * i
        @pl.when(ci < nchunks)
        def _():
            sl = pl.ds(zrows * ci, zrows)
            pltpu.sync_copy(acc.at[sl], out_hbm.at[cid].at[sl])


_GDN = lax.GatherDimensionNumbers(
    offset_dims=(), collapsed_slice_dims=(0,), start_index_map=(0,))


def _lane_take(v, idx):
    return lax.gather(v, idx.reshape(LN, 1), _GDN, (1,),
                      mode=lax.GatherScatterMode.PROMISE_IN_BOUNDS)


def _lane_bcast(v, h):
    return _lane_take(v, lax.iota(jnp.int32, LN) * 0 + h)


def _lane_head(v, j):
    pos = lax.iota(jnp.int32, LN) + LN * j
    one = jnp.ones((LN,), jnp.int32)
    hidx = jnp.zeros((LN,), jnp.int32)
    for b in range(1, 4):
        hidx = hidx + jnp.where(pos >= b * C2, one, 0)
    return _lane_take(v, hidx)


def _mid_body(p_ref, b1_ref, w2_ref, asf_ref, adf_ref, eh_ref,
              src2_ref, asad_ref):
    p = p_ref[...]
    numer = p[0, :, 0:F_IN] + p[1, :, 0:F_IN]
    den = p[0, :, F_IN:F_IN + H] + p[1, :, F_IN:F_IN + H]
    recip = 1.0 / (den + 1e-16)
    rec128 = jnp.dot(recip, eh_ref[...], preferred_element_type=jnp.float32)
    o1 = numer * rec128 + b1_ref[...]
    o1 = jnp.where(o1 > 0, o1, jnp.exp(o1) - 1.0)
    h2 = jnp.dot(o1, w2_ref[...], preferred_element_type=jnp.float32)
    as2 = jnp.dot(h2, asf_ref[...], preferred_element_type=jnp.float32)
    ad2 = jnp.dot(h2, adf_ref[...], preferred_element_type=jnp.float32)
    bn = h2.shape[0]
    z12 = jnp.zeros((bn, 12), jnp.float32)
    halves = []
    for c in range(NC):
        halves.append(jnp.concatenate(
            [h2[:, 160 * c:160 * (c + 1)], as2[:, 4 * c:4 * (c + 1)], z12],
            axis=1))
    src2_ref[...] = jnp.stack(halves, axis=0)
    asad_ref[...] = jnp.concatenate([as2, ad2], axis=1)


def _dst2_body(asad_ref, dst2_ref):
    asad = asad_ref[...]
    as2 = asad[:, 0:H]
    ad2 = asad[:, H:2 * H]
    maxas = jnp.max(as2, axis=0, keepdims=True)
    m2 = jnp.maximum(0.0, maxas + ad2)
    n = asad.shape[0]
    z12 = jnp.zeros((n, 12), jnp.float32)
    halves = []
    for c in range(NC):
        halves.append(jnp.concatenate(
            [ad2[:, 4 * c:4 * (c + 1)], z12, m2[:, 4 * c:4 * (c + 1)], z12],
            axis=1))
    dst2_ref[...] = jnp.stack(halves, axis=0)


def _edge2_body(src_t, dst_t, sidx, didx, out_hbm,
                acc, zbuf, msg, rows_s, rows_d, sv, dv, sem1, sem2):
    n = acc.shape[0]
    cid = lax.axis_index("c")
    sid = lax.axis_index("s")
    e_total = sidx.shape[0]
    e_per_s = e_total // NS
    zrows = zbuf.shape[0]
    nchunks = n // zrows
    per_s = (nchunks + NS - 1) // NS

    def zero_row(r, _):
        for j in range(ACC2_W // LN):
            zbuf[r, pl.ds(LN * j, LN)] = jnp.zeros((LN,), jnp.float32)
        return 0
    lax.fori_loop(0, zrows, zero_row, 0)
    for i in range(per_s):
        ci = sid + NS * i
        @pl.when(ci < nchunks)
        def _():
            pltpu.sync_copy(zbuf, acc.at[pl.ds(zrows * ci, zrows)])
    plsc.subcore_barrier()

    lane = lax.iota(jnp.int32, LN)
    ebase0 = sid * e_per_s

    def chunk_body(t, _):
        base = ebase0 + t * K_E2
        pltpu.sync_copy(sidx.at[pl.ds(base, K_E2)], sv)
        pltpu.sync_copy(didx.at[pl.ds(base, K_E2)], dv)
        cp1 = pltpu.async_copy(src_t.at[cid].at[sv], rows_s, sem1)
        cp2 = pltpu.async_copy(dst_t.at[cid].at[dv], rows_d, sem2)
        cp1.wait()
        cp2.wait()

        def edge_body(k, _):
            asv = rows_s[k, pl.ds(160, LN)]
            adv = rows_d[k, pl.ds(0, LN)]
            mv = rows_d[k, pl.ds(LN, LN)]
            tv = asv + adv
            ev = jnp.maximum(tv, 0.2 * tv)
            w = jnp.exp(ev - mv)
            w = jnp.where(lane < 4, w, 0.0)
            for j in range(160 // LN):
                wl = _lane_head(w, j)
                msg[k, pl.ds(LN * j, LN)] = rows_s[k, pl.ds(LN * j, LN)] * wl
            msg[k, pl.ds(160, LN)] = w
            return 0
        lax.fori_loop(0, K_E2, edge_body, 0)
        pltpu.sync_copy(msg, acc.at[dv], add=True)
        return 0
    lax.fori_loop(0, e_per_s // K_E2, chunk_body, 0)
    plsc.subcore_barrier()

    for i in range(per_s):
        ci = sid + NS * i
        @pl.when(ci < nchunks)
        def _():
            sl = pl.ds(zrows * ci, zrows)
            pltpu.sync_copy(acc.at[sl], out_hbm.at[cid].at[sl])


def _fin_body(a_ref, b2_ref, o_ref):
    a = a_ref[...]
    s = jnp.zeros(o_ref.shape, jnp.float32)
    for g in range(NC):
        for j in range(4):
            numer = a[g, :, C2 * j:C2 * (j + 1)]
            den = a[g, :, 160 + j][:, None]
            s = s + numer / (den + 1e-16)
    o = s * (1.0 / H) + b2_ref[...]
    m = jnp.max(o, axis=1, keepdims=True)
    z = o - m
    lse = jnp.log(jnp.sum(jnp.exp(z), axis=1, keepdims=True))
    o_ref[...] = z - lse


def kernel(x, edge_index, W1, a_s1, a_d1, b1, W2, a_s2, a_d2, b2):
    n = x.shape[0]
    e = edge_index.shape[1]
    src = edge_index[0].astype(jnp.int32)
    dst = edge_index[1].astype(jnp.int32)

    asf1 = _blockdiag(a_s1)
    adf1 = _blockdiag(a_d1)
    asf2 = _blockdiag(a_s2)
    adf2 = _blockdiag(a_d2)
    eh = jnp.repeat(jnp.eye(H, dtype=jnp.float32), C1, axis=1)

    nb = 2000
    grid = n // nb
    src1, asad1 = pl.pallas_call(
        _prep1_body,
        grid=(grid,),
        in_specs=[
            pl.BlockSpec((nb, F_IN), lambda i: (i, 0)),
            pl.BlockSpec((F_IN, H * C1), lambda i: (0, 0)),
            pl.BlockSpec((H * C1, H), lambda i: (0, 0)),
            pl.BlockSpec((H * C1, H), lambda i: (0, 0)),
        ],
        out_specs=[
            pl.BlockSpec((nb, SRC1_W), lambda i: (i, 0)),
            pl.BlockSpec((nb, 2 * H), lambda i: (i, 0)),
        ],
        out_shape=(jax.ShapeDtypeStruct((n, SRC1_W), jnp.float32),
                   jax.ShapeDtypeStruct((n, 2 * H), jnp.float32)),
    )(x, W1, asf1, adf1)

    dst1 = pl.pallas_call(
        _dst1_body,
        out_shape=jax.ShapeDtypeStruct((n, DST1_W), jnp.float32),
    )(asad1)

    edge1 = pl.kernel(
        _edge1_body,
        out_type=jax.ShapeDtypeStruct((NC, n, ACC1_W), jnp.float32),
        mesh=plsc.VectorSubcoreMesh(core_axis_name="c", subcore_axis_name="s"),
        compiler_params=pltpu.CompilerParams(use_tc_tiling_on_sc=False),
        scratch_types=[
            pltpu.VMEM_SHARED((n, ACC1_W), jnp.float32),
            pltpu.VMEM((80, ACC1_W), jnp.float32),
            pltpu.VMEM((K_E, ACC1_W), jnp.float32),
            pltpu.VMEM((K_E, SRC1_W), jnp.float32),
            pltpu.VMEM((K_E, DST1_W), jnp.float32),
            pltpu.VMEM((K_E,), jnp.int32),
            pltpu.VMEM((K_E,), jnp.int32),
            pltpu.SemaphoreType.DMA,
            pltpu.SemaphoreType.DMA,
        ],
    )
    part1 = edge1(src1, dst1, src, dst)

    src2, asad = pl.pallas_call(
        _mid_body,
        grid=(grid,),
        in_specs=[
            pl.BlockSpec((NC, nb, ACC1_W), lambda i: (0, i, 0)),
            pl.BlockSpec((1, F_IN), lambda i: (0, 0)),
            pl.BlockSpec((F_IN, H * C2), lambda i: (0, 0)),
            pl.BlockSpec((H * C2, H), lambda i: (0, 0)),
            pl.BlockSpec((H * C2, H), lambda i: (0, 0)),
            pl.BlockSpec((H, F_IN), lambda i: (0, 0)),
        ],
        out_specs=[
            pl.BlockSpec((NC, nb, SRC2_W), lambda i: (0, i, 0)),
            pl.BlockSpec((nb, 2 * H), lambda i: (i, 0)),
        ],
        out_shape=(jax.ShapeDtypeStruct((NC, n, SRC2_W), jnp.float32),
                   jax.ShapeDtypeStruct((n, 2 * H), jnp.float32)),
    )(part1, b1.reshape(1, F_IN), W2, asf2, adf2, eh)

    dst2 = pl.pallas_call(
        _dst2_body,
        out_shape=jax.ShapeDtypeStruct((NC, n, DST2_W), jnp.float32),
    )(asad)

    edge2 = pl.kernel(
        _edge2_body,
        out_type=jax.ShapeDtypeStruct((NC, n, ACC2_W), jnp.float32),
        mesh=plsc.VectorSubcoreMesh(core_axis_name="c", subcore_axis_name="s"),
        compiler_params=pltpu.CompilerParams(use_tc_tiling_on_sc=False),
        scratch_types=[
            pltpu.VMEM_SHARED((n, ACC2_W), jnp.float32),
            pltpu.VMEM((40, ACC2_W), jnp.float32),
            pltpu.VMEM((K_E2, ACC2_W), jnp.float32),
            pltpu.VMEM((K_E2, SRC2_W), jnp.float32),
            pltpu.VMEM((K_E2, DST2_W), jnp.float32),
            pltpu.VMEM((K_E2,), jnp.int32),
            pltpu.VMEM((K_E2,), jnp.int32),
            pltpu.SemaphoreType.DMA,
            pltpu.SemaphoreType.DMA,
        ],
    )
    part2 = edge2(src2, dst2, src, dst)

    ncls = b2.shape[0]
    out = pl.pallas_call(
        _fin_body,
        grid=(grid,),
        in_specs=[
            pl.BlockSpec((NC, nb, ACC2_W), lambda i: (0, i, 0)),
            pl.BlockSpec((1, ncls), lambda i: (0, 0)),
        ],
        out_specs=pl.BlockSpec((nb, ncls), lambda i: (i, 0)),
        out_shape=jax.ShapeDtypeStruct((n, ncls), jnp.float32),
    )(part2, b2.reshape(1, ncls))
    return out


# speedup vs baseline: 35.9062x; 1.0584x over previous
import functools

import jax
import jax.numpy as jnp
from jax import lax
from jax.experimental import pallas as pl
from jax.experimental.pallas import tpu as pltpu
from jax.experimental.pallas import tpu_sc as plsc

NC = 2
NS = 16
LN = 16

F_IN = 128
H = 8
C1 = 16
C2 = 40
SRC1_W = 144
DST1_W = 16
ACC1_W = 144
SRC2_W = 176
DST2_W = 16
ACC2_W = 168
K_E = 40
K_E2 = 32
ZR = 40


def _blockdiag(a):
    h, c = a.shape
    eye = jnp.eye(h, dtype=a.dtype)
    return (eye[:, None, :] * a[:, :, None]).reshape(h * c, h)


def _prep1_body(x_ref, w1_ref, asf_ref, adf_ref, src_ref, asad_ref):
    n = x_ref.shape[0]
    h1 = jnp.dot(x_ref[...], w1_ref[...], preferred_element_type=jnp.float32)
    as1 = jnp.dot(h1, asf_ref[...], preferred_element_type=jnp.float32)
    ad1 = jnp.dot(h1, adf_ref[...], preferred_element_type=jnp.float32)
    z8 = jnp.zeros((n, H), jnp.float32)
    src_ref[...] = jnp.concatenate([h1, as1, z8], axis=1)
    asad_ref[...] = jnp.concatenate([as1, ad1], axis=1)


def _dst1_body(asad_ref, dst_ref):
    asad = asad_ref[...]
    as1 = asad[:, 0:H]
    ad1 = asad[:, H:2 * H]
    maxas = jnp.max(as1, axis=0, keepdims=True)
    m1 = jnp.maximum(0.0, maxas + ad1)
    dst_ref[...] = jnp.concatenate([ad1, m1], axis=1)


def _zero_acc(acc, zbuf, sid, width):
    n = acc.shape[0]
    zrows = zbuf.shape[0]
    nchunks = n // zrows
    per_s = (nchunks + NS - 1) // NS
    nvec = (width + LN - 1) // LN

    def zero_row(r, _):
        for j in range(nvec):
            off = min(LN * j, width - LN)
            zbuf[r, pl.ds(off, LN)] = jnp.zeros((LN,), jnp.float32)
        return 0
    lax.fori_loop(0, zrows, zero_row, 0)
    for i in range(per_s):
        ci = sid + NS * i
        @pl.when(ci < nchunks)
        def _():
            pltpu.sync_copy(zbuf, acc.at[pl.ds(zrows * ci, zrows)])


def _write_acc(acc, out_plane, zbuf, sid):
    n = acc.shape[0]
    zrows = zbuf.shape[0]
    nchunks = n // zrows
    per_s = (nchunks + NS - 1) // NS
    for i in range(per_s):
        ci = sid + NS * i
        @pl.when(ci < nchunks)
        def _():
            sl = pl.ds(zrows * ci, zrows)
            pltpu.sync_copy(acc.at[sl], out_plane.at[sl])


def _edge1_body(src_t, dst_t, sidx, didx, out_hbm,
                acc, zbuf, msg,
                rs0, rs1, rd0, rd1, sv0, sv1, dv0, dv1,
                semg0, semg1, semi0, semi1):
    n = acc.shape[0]
    cid = lax.axis_index("c")
    sid = lax.axis_index("s")
    wid = cid * NS + sid
    e_per_w = sidx.shape[0] // (NC * NS)
    nt = e_per_w // K_E
    ebase0 = wid * e_per_w

    _zero_acc(acc, zbuf, sid, ACC1_W)
    plsc.subcore_barrier()

    lane = lax.iota(jnp.int32, LN)
    midx = jnp.minimum(lane + H, LN - 1)
    rs = [rs0, rs1]
    rd = [rd0, rd1]
    sv = [sv0, sv1]
    dv = [dv0, dv1]
    semg = [semg0, semg1]
    semi = [semi0, semi1]

    def issue_idx(b, t):
        base = ebase0 + t * K_E
        pltpu.async_copy(sidx.at[pl.ds(base, K_E)], sv[b], semi[b])
        pltpu.async_copy(didx.at[pl.ds(base, K_E)], dv[b], semi[b])

    def wait_idx(b):
        pltpu.make_async_copy(sidx.at[pl.ds(0, K_E)], sv[b], semi[b]).wait()
        pltpu.make_async_copy(didx.at[pl.ds(0, K_E)], dv[b], semi[b]).wait()

    def issue_gather(b):
        pltpu.async_copy(src_t.at[sv[b]], rs[b], semg[b])
        pltpu.async_copy(dst_t.at[dv[b]], rd[b], semg[b])

    def wait_gather(b):
        pltpu.make_async_copy(src_t.at[sv[b]], rs[b], semg[b]).wait()
        pltpu.make_async_copy(dst_t.at[dv[b]], rd[b], semg[b]).wait()

    def process(b):
        rows_s = rs[b]
        rows_d = rd[b]

        def edge_body(k, _):
            asv = rows_s[k, pl.ds(F_IN, LN)]
            adv = rows_d[k, pl.ds(0, LN)]
            mv = _lane_take(adv, midx)
            tv = asv + adv
            ev = jnp.maximum(tv, 0.2 * tv)
            w = jnp.exp(ev - mv)
            w = jnp.where(lane < H, w, 0.0)
            for hh in range(H):
                wh = _lane_bcast(w, hh)
                msg[k, pl.ds(LN * hh, LN)] = rows_s[k, pl.ds(LN * hh, LN)] * wh
            msg[k, pl.ds(F_IN, LN)] = w
            return 0
        lax.fori_loop(0, K_E, edge_body, 0)
        pltpu.sync_copy(msg, acc.at[dv[b]], add=True)

    issue_idx(0, 0)
    wait_idx(0)
    issue_gather(0)
    issue_idx(1, 1)

    def pair_body(u, _):
        t0 = 2 * u
        wait_gather(0)
        wait_idx(1)
        issue_gather(1)
        process(0)
        @pl.when(t0 + 2 < nt)
        def _():
            issue_idx(0, t0 + 2)
        wait_gather(1)
        @pl.when(t0 + 2 < nt)
        def _():
            wait_idx(0)
            issue_gather(0)
        process(1)
        @pl.when(t0 + 3 < nt)
        def _():
            issue_idx(1, t0 + 3)
        return 0
    lax.fori_loop(0, nt // 2, pair_body, 0)
    plsc.subcore_barrier()
    _write_acc(acc, out_hbm.at[cid], zbuf, sid)


_GDN = lax.GatherDimensionNumbers(
    offset_dims=(), collapsed_slice_dims=(0,), start_index_map=(0,))


def _lane_take(v, idx):
    return lax.gather(v, idx.reshape(LN, 1), _GDN, (1,),
                      mode=lax.GatherScatterMode.PROMISE_IN_BOUNDS)


def _lane_bcast(v, h):
    return _lane_take(v, lax.iota(jnp.int32, LN) * 0 + h)


def _lane_head(v, j):
    pos = lax.iota(jnp.int32, LN) + LN * j
    one = jnp.ones((LN,), jnp.int32)
    hidx = jnp.zeros((LN,), jnp.int32)
    for b in range(1, 4):
        hidx = hidx + jnp.where(pos >= b * C2, one, 0)
    return _lane_take(v, hidx)


def _mid_body(p_ref, b1_ref, w2_ref, asf_ref, adf_ref, eh_ref,
              src2_ref, asad_ref):
    p = p_ref[...]
    numer = p[0, :, 0:F_IN] + p[1, :, 0:F_IN]
    den = p[0, :, F_IN:F_IN + H] + p[1, :, F_IN:F_IN + H]
    recip = 1.0 / (den + 1e-16)
    rec128 = jnp.dot(recip, eh_ref[...], preferred_element_type=jnp.float32)
    o1 = numer * rec128 + b1_ref[...]
    o1 = jnp.where(o1 > 0, o1, jnp.exp(o1) - 1.0)
    h2 = jnp.dot(o1, w2_ref[...], preferred_element_type=jnp.float32)
    as2 = jnp.dot(h2, asf_ref[...], preferred_element_type=jnp.float32)
    ad2 = jnp.dot(h2, adf_ref[...], preferred_element_type=jnp.float32)
    bn = h2.shape[0]
    z12 = jnp.zeros((bn, 12), jnp.float32)
    halves = []
    for c in range(NC):
        halves.append(jnp.concatenate(
            [h2[:, 160 * c:160 * (c + 1)], as2[:, 4 * c:4 * (c + 1)], z12],
            axis=1))
    src2_ref[...] = jnp.stack(halves, axis=0)
    asad_ref[...] = jnp.concatenate([as2, ad2], axis=1)


def _dst2_body(asad_ref, dst2_ref):
    asad = asad_ref[...]
    as2 = asad[:, 0:H]
    ad2 = asad[:, H:2 * H]
    maxas = jnp.max(as2, axis=0, keepdims=True)
    m2 = jnp.maximum(0.0, maxas + ad2)
    n = asad.shape[0]
    z8 = jnp.zeros((n, H), jnp.float32)
    halves = []
    for c in range(NC):
        halves.append(jnp.concatenate(
            [ad2[:, 4 * c:4 * (c + 1)], m2[:, 4 * c:4 * (c + 1)], z8],
            axis=1))
    dst2_ref[...] = jnp.stack(halves, axis=0)


def _edge2_body(src_t, dst_t, sidx, didx, out_hbm,
                acc, zbuf, msg,
                rs0, rs1, rd0, rd1, sv0, sv1, dv0, dv1,
                semg0, semg1, semi0, semi1):
    n = acc.shape[0]
    cid = lax.axis_index("c")
    sid = lax.axis_index("s")
    e_per_s = sidx.shape[0] // NS
    nt = e_per_s // K_E2
    ebase0 = sid * e_per_s

    _zero_acc(acc, zbuf, sid, ACC2_W)
    plsc.subcore_barrier()

    lane = lax.iota(jnp.int32, LN)
    midx = jnp.minimum(lane + 4, LN - 1)
    hiidx = jnp.minimum(lane + H, LN - 1)
    loidx = jnp.maximum(lane - H, 0)
    rs = [rs0, rs1]
    rd = [rd0, rd1]
    sv = [sv0, sv1]
    dv = [dv0, dv1]
    semg = [semg0, semg1]
    semi = [semi0, semi1]

    def issue_idx(b, t):
        base = ebase0 + t * K_E2
        pltpu.async_copy(sidx.at[pl.ds(base, K_E2)], sv[b], semi[b])
        pltpu.async_copy(didx.at[pl.ds(base, K_E2)], dv[b], semi[b])

    def wait_idx(b):
        pltpu.make_async_copy(sidx.at[pl.ds(0, K_E2)], sv[b], semi[b]).wait()
        pltpu.make_async_copy(didx.at[pl.ds(0, K_E2)], dv[b], semi[b]).wait()

    def issue_gather(b):
        pltpu.async_copy(src_t.at[cid].at[sv[b]], rs[b], semg[b])
        pltpu.async_copy(dst_t.at[cid].at[dv[b]], rd[b], semg[b])

    def wait_gather(b):
        pltpu.make_async_copy(src_t.at[cid].at[sv[b]], rs[b], semg[b]).wait()
        pltpu.make_async_copy(dst_t.at[cid].at[dv[b]], rd[b], semg[b]).wait()

    def process(b):
        rows_s = rs[b]
        rows_d = rd[b]

        def edge_body(k, _):
            asv = rows_s[k, pl.ds(160, LN)]
            adv = rows_d[k, pl.ds(0, LN)]
            mv = _lane_take(adv, midx)
            tv = asv + adv
            ev = jnp.maximum(tv, 0.2 * tv)
            w = jnp.exp(ev - mv)
            w = jnp.where(lane < 4, w, 0.0)
            vj9 = None
            for j in range(160 // LN):
                wl = _lane_head(w, j)
                vj = rows_s[k, pl.ds(LN * j, LN)] * wl
                msg[k, pl.ds(LN * j, LN)] = vj
                vj9 = vj
            comb = jnp.where(lane < H, _lane_take(vj9, hiidx),
                             _lane_take(w, loidx))
            msg[k, pl.ds(152, LN)] = comb
            return 0
        lax.fori_loop(0, K_E2, edge_body, 0)
        pltpu.sync_copy(msg, acc.at[dv[b]], add=True)

    issue_idx(0, 0)
    wait_idx(0)
    issue_gather(0)
    issue_idx(1, 1)

    def pair_body(u, _):
        t0 = 2 * u
        wait_gather(0)
        wait_idx(1)
        issue_gather(1)
        process(0)
        @pl.when(t0 + 2 < nt)
        def _():
            issue_idx(0, t0 + 2)
        wait_gather(1)
        @pl.when(t0 + 2 < nt)
        def _():
            wait_idx(0)
            issue_gather(0)
        process(1)
        @pl.when(t0 + 3 < nt)
        def _():
            issue_idx(1, t0 + 3)
        return 0
    lax.fori_loop(0, nt // 2, pair_body, 0)
    if nt % 2 == 1:
        wait_gather(0)
        process(0)
    plsc.subcore_barrier()
    _write_acc(acc, out_hbm.at[cid], zbuf, sid)


def _fin_body(a_ref, b2_ref, o_ref):
    a = a_ref[...]
    s = jnp.zeros(o_ref.shape, jnp.float32)
    for g in range(NC):
        for j in range(4):
            numer = a[g, :, C2 * j:C2 * (j + 1)]
            den = a[g, :, 160 + j][:, None]
            s = s + numer / (den + 1e-16)
    o = s * (1.0 / H) + b2_ref[...]
    m = jnp.max(o, axis=1, keepdims=True)
    z = o - m
    lse = jnp.log(jnp.sum(jnp.exp(z), axis=1, keepdims=True))
    o_ref[...] = z - lse


def kernel(x, edge_index, W1, a_s1, a_d1, b1, W2, a_s2, a_d2, b2):
    n = x.shape[0]
    e = edge_index.shape[1]
    src = edge_index[0].astype(jnp.int32)
    dst = edge_index[1].astype(jnp.int32)

    asf1 = _blockdiag(a_s1)
    adf1 = _blockdiag(a_d1)
    asf2 = _blockdiag(a_s2)
    adf2 = _blockdiag(a_d2)
    eh = jnp.repeat(jnp.eye(H, dtype=jnp.float32), C1, axis=1)

    nb = 2000
    grid = n // nb
    src1, asad1 = pl.pallas_call(
        _prep1_body,
        grid=(grid,),
        in_specs=[
            pl.BlockSpec((nb, F_IN), lambda i: (i, 0)),
            pl.BlockSpec((F_IN, H * C1), lambda i: (0, 0)),
            pl.BlockSpec((H * C1, H), lambda i: (0, 0)),
            pl.BlockSpec((H * C1, H), lambda i: (0, 0)),
        ],
        out_specs=[
            pl.BlockSpec((nb, SRC1_W), lambda i: (i, 0)),
            pl.BlockSpec((nb, 2 * H), lambda i: (i, 0)),
        ],
        out_shape=(jax.ShapeDtypeStruct((n, SRC1_W), jnp.float32),
                   jax.ShapeDtypeStruct((n, 2 * H), jnp.float32)),
    )(x, W1, asf1, adf1)

    dst1 = pl.pallas_call(
        _dst1_body,
        out_shape=jax.ShapeDtypeStruct((n, DST1_W), jnp.float32),
    )(asad1)

    edge1 = pl.kernel(
        _edge1_body,
        out_type=jax.ShapeDtypeStruct((NC, n, ACC1_W), jnp.float32),
        mesh=plsc.VectorSubcoreMesh(core_axis_name="c", subcore_axis_name="s"),
        compiler_params=pltpu.CompilerParams(use_tc_tiling_on_sc=False),
        scratch_types=[
            pltpu.VMEM_SHARED((n, ACC1_W), jnp.float32),
            pltpu.VMEM((ZR, ACC1_W), jnp.float32),
            pltpu.VMEM((K_E, ACC1_W), jnp.float32),
            pltpu.VMEM((K_E, SRC1_W), jnp.float32),
            pltpu.VMEM((K_E, SRC1_W), jnp.float32),
            pltpu.VMEM((K_E, DST1_W), jnp.float32),
            pltpu.VMEM((K_E, DST1_W), jnp.float32),
            pltpu.VMEM((K_E,), jnp.int32),
            pltpu.VMEM((K_E,), jnp.int32),
            pltpu.VMEM((K_E,), jnp.int32),
            pltpu.VMEM((K_E,), jnp.int32),
            pltpu.SemaphoreType.DMA,
            pltpu.SemaphoreType.DMA,
            pltpu.SemaphoreType.DMA,
            pltpu.SemaphoreType.DMA,
        ],
    )
    part1 = edge1(src1, dst1, src, dst)

    src2, asad = pl.pallas_call(
        _mid_body,
        grid=(grid,),
        in_specs=[
            pl.BlockSpec((NC, nb, ACC1_W), lambda i: (0, i, 0)),
            pl.BlockSpec((1, F_IN), lambda i: (0, 0)),
            pl.BlockSpec((F_IN, H * C2), lambda i: (0, 0)),
            pl.BlockSpec((H * C2, H), lambda i: (0, 0)),
            pl.BlockSpec((H * C2, H), lambda i: (0, 0)),
            pl.BlockSpec((H, F_IN), lambda i: (0, 0)),
        ],
        out_specs=[
            pl.BlockSpec((NC, nb, SRC2_W), lambda i: (0, i, 0)),
            pl.BlockSpec((nb, 2 * H), lambda i: (i, 0)),
        ],
        out_shape=(jax.ShapeDtypeStruct((NC, n, SRC2_W), jnp.float32),
                   jax.ShapeDtypeStruct((n, 2 * H), jnp.float32)),
    )(part1, b1.reshape(1, F_IN), W2, asf2, adf2, eh)

    dst2 = pl.pallas_call(
        _dst2_body,
        out_shape=jax.ShapeDtypeStruct((NC, n, DST2_W), jnp.float32),
    )(asad)

    edge2 = pl.kernel(
        _edge2_body,
        out_type=jax.ShapeDtypeStruct((NC, n, ACC2_W), jnp.float32),
        mesh=plsc.VectorSubcoreMesh(core_axis_name="c", subcore_axis_name="s"),
        compiler_params=pltpu.CompilerParams(use_tc_tiling_on_sc=False),
        scratch_types=[
            pltpu.VMEM_SHARED((n, ACC2_W), jnp.float32),
            pltpu.VMEM((ZR, ACC2_W), jnp.float32),
            pltpu.VMEM((K_E2, ACC2_W), jnp.float32),
            pltpu.VMEM((K_E2, SRC2_W), jnp.float32),
            pltpu.VMEM((K_E2, SRC2_W), jnp.float32),
            pltpu.VMEM((K_E2, DST2_W), jnp.float32),
            pltpu.VMEM((K_E2, DST2_W), jnp.float32),
            pltpu.VMEM((K_E2,), jnp.int32),
            pltpu.VMEM((K_E2,), jnp.int32),
            pltpu.VMEM((K_E2,), jnp.int32),
            pltpu.VMEM((K_E2,), jnp.int32),
            pltpu.SemaphoreType.DMA,
            pltpu.SemaphoreType.DMA,
            pltpu.SemaphoreType.DMA,
            pltpu.SemaphoreType.DMA,
        ],
    )
    part2 = edge2(src2, dst2, src, dst)

    ncls = b2.shape[0]
    out = pl.pallas_call(
        _fin_body,
        grid=(grid,),
        in_specs=[
            pl.BlockSpec((NC, nb, ACC2_W), lambda i: (0, i, 0)),
            pl.BlockSpec((1, ncls), lambda i: (0, 0)),
        ],
        out_specs=pl.BlockSpec((nb, ncls), lambda i: (i, 0)),
        out_shape=jax.ShapeDtypeStruct((n, ncls), jnp.float32),
    )(part2, b2.reshape(1, ncls))
    return out


# speedup vs baseline: 72.1875x; 2.0104x over previous
import functools

import jax
import jax.numpy as jnp
from jax import lax
from jax.experimental import pallas as pl
from jax.experimental.pallas import tpu as pltpu
from jax.experimental.pallas import tpu_sc as plsc

NC = 2
NS = 16
LN = 16

F_IN = 128
H = 8
C1 = 16
C2 = 40
SRC1_W = 144
DST1_W = 16
ACC1_W = 144
SRC2_W = 176
DST2_W = 16
ACC2_W = 168
K_E = 40
K_E2 = 32
ZR = 40


def _blockdiag(a):
    h, c = a.shape
    eye = jnp.eye(h, dtype=a.dtype)
    return (eye[:, None, :] * a[:, :, None]).reshape(h * c, h)


def _prep1_body(x_ref, w1_ref, asf_ref, adf_ref, src_ref, asad_ref):
    n = x_ref.shape[0]
    h1 = jnp.dot(x_ref[...], w1_ref[...], preferred_element_type=jnp.float32)
    as1 = jnp.dot(h1, asf_ref[...], preferred_element_type=jnp.float32)
    ad1 = jnp.dot(h1, adf_ref[...], preferred_element_type=jnp.float32)
    z8 = jnp.zeros((n, H), jnp.float32)
    src_ref[...] = jnp.concatenate([h1, as1, z8], axis=1)
    asad_ref[...] = jnp.concatenate([as1, ad1], axis=1)


def _dst1_body(asad_ref, dst_ref):
    asad = asad_ref[...]
    as1 = asad[:, 0:H]
    ad1 = asad[:, H:2 * H]
    maxas = jnp.max(as1, axis=0, keepdims=True)
    m1 = jnp.maximum(0.0, maxas + ad1)
    dst_ref[...] = jnp.concatenate([ad1, m1], axis=1)


def _zero_acc(acc, zbuf, sid, width):
    n = acc.shape[0]
    zrows = zbuf.shape[0]
    nchunks = n // zrows
    per_s = (nchunks + NS - 1) // NS
    nvec = (width + LN - 1) // LN

    def zero_row(r, _):
        for j in range(nvec):
            off = min(LN * j, width - LN)
            zbuf[r, pl.ds(off, LN)] = jnp.zeros((LN,), jnp.float32)
        return 0
    lax.fori_loop(0, zrows, zero_row, 0)
    for i in range(per_s):
        ci = sid + NS * i
        @pl.when(ci < nchunks)
        def _():
            pltpu.sync_copy(zbuf, acc.at[pl.ds(zrows * ci, zrows)])


def _write_acc(acc, out_plane, zbuf, sid):
    n = acc.shape[0]
    zrows = zbuf.shape[0]
    nchunks = n // zrows
    per_s = (nchunks + NS - 1) // NS
    for i in range(per_s):
        ci = sid + NS * i
        @pl.when(ci < nchunks)
        def _():
            sl = pl.ds(zrows * ci, zrows)
            pltpu.sync_copy(acc.at[sl], out_plane.at[sl])


def _edge1_body(src_t, dst_t, sidx, didx, out_hbm,
                acc, zbuf, msg,
                rs0, rs1, rd0, rd1, sv0, sv1, dv0, dv1,
                semg0, semg1, semi0, semi1):
    n = acc.shape[0]
    cid = lax.axis_index("c")
    sid = lax.axis_index("s")
    wid = cid * NS + sid
    e_per_w = sidx.shape[0] // (NC * NS)
    nt = e_per_w // K_E
    ebase0 = wid * e_per_w

    _zero_acc(acc, zbuf, sid, ACC1_W)
    plsc.subcore_barrier()

    lane = lax.iota(jnp.int32, LN)
    midx = jnp.minimum(lane + H, LN - 1)
    rs = [rs0, rs1]
    rd = [rd0, rd1]
    sv = [sv0, sv1]
    dv = [dv0, dv1]
    semg = [semg0, semg1]
    semi = [semi0, semi1]

    def issue_idx(b, t):
        base = ebase0 + t * K_E
        pltpu.async_copy(sidx.at[pl.ds(base, K_E)], sv[b], semi[b])
        pltpu.async_copy(didx.at[pl.ds(base, K_E)], dv[b], semi[b])

    def wait_idx(b):
        pltpu.make_async_copy(sidx.at[pl.ds(0, K_E)], sv[b], semi[b]).wait()
        pltpu.make_async_copy(didx.at[pl.ds(0, K_E)], dv[b], semi[b]).wait()

    def issue_gather(b):
        pltpu.async_copy(src_t.at[sv[b]], rs[b], semg[b])
        pltpu.async_copy(dst_t.at[dv[b]], rd[b], semg[b])

    def wait_gather(b):
        pltpu.make_async_copy(src_t.at[sv[b]], rs[b], semg[b]).wait()
        pltpu.make_async_copy(dst_t.at[dv[b]], rd[b], semg[b]).wait()

    def process(b):
        rows_s = rs[b]
        rows_d = rd[b]

        @plsc.parallel_loop(0, K_E, unroll=4)
        def edge_body(k):
            asv = rows_s[k, pl.ds(F_IN, LN)]
            adv = rows_d[k, pl.ds(0, LN)]
            mv = _lane_take(adv, midx)
            tv = asv + adv
            ev = jnp.maximum(tv, 0.2 * tv)
            w = jnp.exp(ev - mv)
            w = jnp.where(lane < H, w, 0.0)
            for hh in range(H):
                wh = _lane_bcast(w, hh)
                msg[k, pl.ds(LN * hh, LN)] = rows_s[k, pl.ds(LN * hh, LN)] * wh
            msg[k, pl.ds(F_IN, LN)] = w
        pltpu.sync_copy(msg, acc.at[dv[b]], add=True)

    issue_idx(0, 0)
    wait_idx(0)
    issue_gather(0)
    issue_idx(1, 1)

    def pair_body(u, _):
        t0 = 2 * u
        wait_gather(0)
        wait_idx(1)
        issue_gather(1)
        process(0)
        @pl.when(t0 + 2 < nt)
        def _():
            issue_idx(0, t0 + 2)
        wait_gather(1)
        @pl.when(t0 + 2 < nt)
        def _():
            wait_idx(0)
            issue_gather(0)
        process(1)
        @pl.when(t0 + 3 < nt)
        def _():
            issue_idx(1, t0 + 3)
        return 0
    lax.fori_loop(0, nt // 2, pair_body, 0)
    plsc.subcore_barrier()
    _write_acc(acc, out_hbm.at[cid], zbuf, sid)


_GDN = lax.GatherDimensionNumbers(
    offset_dims=(), collapsed_slice_dims=(0,), start_index_map=(0,))


def _lane_take(v, idx):
    return lax.gather(v, idx.reshape(LN, 1), _GDN, (1,),
                      mode=lax.GatherScatterMode.PROMISE_IN_BOUNDS)


def _lane_bcast(v, h):
    return _lane_take(v, lax.iota(jnp.int32, LN) * 0 + h)


def _lane_head(v, j):
    pos = lax.iota(jnp.int32, LN) + LN * j
    one = jnp.ones((LN,), jnp.int32)
    hidx = jnp.zeros((LN,), jnp.int32)
    for b in range(1, 4):
        hidx = hidx + jnp.where(pos >= b * C2, one, 0)
    return _lane_take(v, hidx)


def _mid_body(p_ref, b1_ref, w2_ref, asf_ref, adf_ref, eh_ref,
              src2_ref, asad_ref):
    p = p_ref[...]
    numer = p[0, :, 0:F_IN] + p[1, :, 0:F_IN]
    den = p[0, :, F_IN:F_IN + H] + p[1, :, F_IN:F_IN + H]
    recip = 1.0 / (den + 1e-16)
    rec128 = jnp.dot(recip, eh_ref[...], preferred_element_type=jnp.float32)
    o1 = numer * rec128 + b1_ref[...]
    o1 = jnp.where(o1 > 0, o1, jnp.exp(o1) - 1.0)
    h2 = jnp.dot(o1, w2_ref[...], preferred_element_type=jnp.float32)
    as2 = jnp.dot(h2, asf_ref[...], preferred_element_type=jnp.float32)
    ad2 = jnp.dot(h2, adf_ref[...], preferred_element_type=jnp.float32)
    bn = h2.shape[0]
    z12 = jnp.zeros((bn, 12), jnp.float32)
    halves = []
    for c in range(NC):
        halves.append(jnp.concatenate(
            [h2[:, 160 * c:160 * (c + 1)], as2[:, 4 * c:4 * (c + 1)], z12],
            axis=1))
    src2_ref[...] = jnp.stack(halves, axis=0)
    asad_ref[...] = jnp.concatenate([as2, ad2], axis=1)


def _dst2_body(asad_ref, dst2_ref):
    asad = asad_ref[...]
    as2 = asad[:, 0:H]
    ad2 = asad[:, H:2 * H]
    maxas = jnp.max(as2, axis=0, keepdims=True)
    m2 = jnp.maximum(0.0, maxas + ad2)
    n = asad.shape[0]
    z8 = jnp.zeros((n, H), jnp.float32)
    halves = []
    for c in range(NC):
        halves.append(jnp.concatenate(
            [ad2[:, 4 * c:4 * (c + 1)], m2[:, 4 * c:4 * (c + 1)], z8],
            axis=1))
    dst2_ref[...] = jnp.stack(halves, axis=0)


def _edge2_body(src_t, dst_t, sidx, didx, out_hbm,
                acc, zbuf, msg,
                rs0, rs1, rd0, rd1, sv0, sv1, dv0, dv1,
                semg0, semg1, semi0, semi1):
    n = acc.shape[0]
    cid = lax.axis_index("c")
    sid = lax.axis_index("s")
    e_per_s = sidx.shape[0] // NS
    nt = e_per_s // K_E2
    ebase0 = sid * e_per_s

    _zero_acc(acc, zbuf, sid, ACC2_W)
    plsc.subcore_barrier()

    lane = lax.iota(jnp.int32, LN)
    midx = jnp.minimum(lane + 4, LN - 1)
    hiidx = jnp.minimum(lane + H, LN - 1)
    loidx = jnp.maximum(lane - H, 0)
    rs = [rs0, rs1]
    rd = [rd0, rd1]
    sv = [sv0, sv1]
    dv = [dv0, dv1]
    semg = [semg0, semg1]
    semi = [semi0, semi1]

    def issue_idx(b, t):
        base = ebase0 + t * K_E2
        pltpu.async_copy(sidx.at[pl.ds(base, K_E2)], sv[b], semi[b])
        pltpu.async_copy(didx.at[pl.ds(base, K_E2)], dv[b], semi[b])

    def wait_idx(b):
        pltpu.make_async_copy(sidx.at[pl.ds(0, K_E2)], sv[b], semi[b]).wait()
        pltpu.make_async_copy(didx.at[pl.ds(0, K_E2)], dv[b], semi[b]).wait()

    def issue_gather(b):
        pltpu.async_copy(src_t.at[cid].at[sv[b]], rs[b], semg[b])
        pltpu.async_copy(dst_t.at[cid].at[dv[b]], rd[b], semg[b])

    def wait_gather(b):
        pltpu.make_async_copy(src_t.at[cid].at[sv[b]], rs[b], semg[b]).wait()
        pltpu.make_async_copy(dst_t.at[cid].at[dv[b]], rd[b], semg[b]).wait()

    def process(b):
        rows_s = rs[b]
        rows_d = rd[b]

        @plsc.parallel_loop(0, K_E2, unroll=4)
        def edge_body(k):
            asv = rows_s[k, pl.ds(160, LN)]
            adv = rows_d[k, pl.ds(0, LN)]
            mv = _lane_take(adv, midx)
            tv = asv + adv
            ev = jnp.maximum(tv, 0.2 * tv)
            w = jnp.exp(ev - mv)
            w = jnp.where(lane < 4, w, 0.0)
            vj9 = None
            for j in range(160 // LN):
                wl = _lane_head(w, j)
                vj = rows_s[k, pl.ds(LN * j, LN)] * wl
                msg[k, pl.ds(LN * j, LN)] = vj
                vj9 = vj
            comb = jnp.where(lane < H, _lane_take(vj9, hiidx),
                             _lane_take(w, loidx))
            msg[k, pl.ds(152, LN)] = comb
        pltpu.sync_copy(msg, acc.at[dv[b]], add=True)

    issue_idx(0, 0)
    wait_idx(0)
    issue_gather(0)
    issue_idx(1, 1)

    def pair_body(u, _):
        t0 = 2 * u
        wait_gather(0)
        wait_idx(1)
        issue_gather(1)
        process(0)
        @pl.when(t0 + 2 < nt)
        def _():
            issue_idx(0, t0 + 2)
        wait_gather(1)
        @pl.when(t0 + 2 < nt)
        def _():
            wait_idx(0)
            issue_gather(0)
        process(1)
        @pl.when(t0 + 3 < nt)
        def _():
            issue_idx(1, t0 + 3)
        return 0
    lax.fori_loop(0, nt // 2, pair_body, 0)
    if nt % 2 == 1:
        wait_gather(0)
        process(0)
    plsc.subcore_barrier()
    _write_acc(acc, out_hbm.at[cid], zbuf, sid)


def _fin_body(a_ref, b2_ref, o_ref):
    a = a_ref[...]
    s = jnp.zeros(o_ref.shape, jnp.float32)
    for g in range(NC):
        for j in range(4):
            numer = a[g, :, C2 * j:C2 * (j + 1)]
            den = a[g, :, 160 + j][:, None]
            s = s + numer / (den + 1e-16)
    o = s * (1.0 / H) + b2_ref[...]
    m = jnp.max(o, axis=1, keepdims=True)
    z = o - m
    lse = jnp.log(jnp.sum(jnp.exp(z), axis=1, keepdims=True))
    o_ref[...] = z - lse


def kernel(x, edge_index, W1, a_s1, a_d1, b1, W2, a_s2, a_d2, b2):
    n = x.shape[0]
    e = edge_index.shape[1]
    src = edge_index[0].astype(jnp.int32)
    dst = edge_index[1].astype(jnp.int32)

    asf1 = _blockdiag(a_s1)
    adf1 = _blockdiag(a_d1)
    asf2 = _blockdiag(a_s2)
    adf2 = _blockdiag(a_d2)
    eh = jnp.repeat(jnp.eye(H, dtype=jnp.float32), C1, axis=1)

    nb = 2000
    grid = n // nb
    src1, asad1 = pl.pallas_call(
        _prep1_body,
        grid=(grid,),
        in_specs=[
            pl.BlockSpec((nb, F_IN), lambda i: (i, 0)),
            pl.BlockSpec((F_IN, H * C1), lambda i: (0, 0)),
            pl.BlockSpec((H * C1, H), lambda i: (0, 0)),
            pl.BlockSpec((H * C1, H), lambda i: (0, 0)),
        ],
        out_specs=[
            pl.BlockSpec((nb, SRC1_W), lambda i: (i, 0)),
            pl.BlockSpec((nb, 2 * H), lambda i: (i, 0)),
        ],
        out_shape=(jax.ShapeDtypeStruct((n, SRC1_W), jnp.float32),
                   jax.ShapeDtypeStruct((n, 2 * H), jnp.float32)),
    )(x, W1, asf1, adf1)

    dst1 = pl.pallas_call(
        _dst1_body,
        out_shape=jax.ShapeDtypeStruct((n, DST1_W), jnp.float32),
    )(asad1)

    edge1 = pl.kernel(
        _edge1_body,
        out_type=jax.ShapeDtypeStruct((NC, n, ACC1_W), jnp.float32),
        mesh=plsc.VectorSubcoreMesh(core_axis_name="c", subcore_axis_name="s"),
        compiler_params=pltpu.CompilerParams(use_tc_tiling_on_sc=False),
        scratch_types=[
            pltpu.VMEM_SHARED((n, ACC1_W), jnp.float32),
            pltpu.VMEM((ZR, ACC1_W), jnp.float32),
            pltpu.VMEM((K_E, ACC1_W), jnp.float32),
            pltpu.VMEM((K_E, SRC1_W), jnp.float32),
            pltpu.VMEM((K_E, SRC1_W), jnp.float32),
            pltpu.VMEM((K_E, DST1_W), jnp.float32),
            pltpu.VMEM((K_E, DST1_W), jnp.float32),
            pltpu.VMEM((K_E,), jnp.int32),
            pltpu.VMEM((K_E,), jnp.int32),
            pltpu.VMEM((K_E,), jnp.int32),
            pltpu.VMEM((K_E,), jnp.int32),
            pltpu.SemaphoreType.DMA,
            pltpu.SemaphoreType.DMA,
            pltpu.SemaphoreType.DMA,
            pltpu.SemaphoreType.DMA,
        ],
    )
    part1 = edge1(src1, dst1, src, dst)

    src2, asad = pl.pallas_call(
        _mid_body,
        grid=(grid,),
        in_specs=[
            pl.BlockSpec((NC, nb, ACC1_W), lambda i: (0, i, 0)),
            pl.BlockSpec((1, F_IN), lambda i: (0, 0)),
            pl.BlockSpec((F_IN, H * C2), lambda i: (0, 0)),
            pl.BlockSpec((H * C2, H), lambda i: (0, 0)),
            pl.BlockSpec((H * C2, H), lambda i: (0, 0)),
            pl.BlockSpec((H, F_IN), lambda i: (0, 0)),
        ],
        out_specs=[
            pl.BlockSpec((NC, nb, SRC2_W), lambda i: (0, i, 0)),
            pl.BlockSpec((nb, 2 * H), lambda i: (i, 0)),
        ],
        out_shape=(jax.ShapeDtypeStruct((NC, n, SRC2_W), jnp.float32),
                   jax.ShapeDtypeStruct((n, 2 * H), jnp.float32)),
    )(part1, b1.reshape(1, F_IN), W2, asf2, adf2, eh)

    dst2 = pl.pallas_call(
        _dst2_body,
        out_shape=jax.ShapeDtypeStruct((NC, n, DST2_W), jnp.float32),
    )(asad)

    edge2 = pl.kernel(
        _edge2_body,
        out_type=jax.ShapeDtypeStruct((NC, n, ACC2_W), jnp.float32),
        mesh=plsc.VectorSubcoreMesh(core_axis_name="c", subcore_axis_name="s"),
        compiler_params=pltpu.CompilerParams(use_tc_tiling_on_sc=False),
        scratch_types=[
            pltpu.VMEM_SHARED((n, ACC2_W), jnp.float32),
            pltpu.VMEM((ZR, ACC2_W), jnp.float32),
            pltpu.VMEM((K_E2, ACC2_W), jnp.float32),
            pltpu.VMEM((K_E2, SRC2_W), jnp.float32),
            pltpu.VMEM((K_E2, SRC2_W), jnp.float32),
            pltpu.VMEM((K_E2, DST2_W), jnp.float32),
            pltpu.VMEM((K_E2, DST2_W), jnp.float32),
            pltpu.VMEM((K_E2,), jnp.int32),
            pltpu.VMEM((K_E2,), jnp.int32),
            pltpu.VMEM((K_E2,), jnp.int32),
            pltpu.VMEM((K_E2,), jnp.int32),
            pltpu.SemaphoreType.DMA,
            pltpu.SemaphoreType.DMA,
            pltpu.SemaphoreType.DMA,
            pltpu.SemaphoreType.DMA,
        ],
    )
    part2 = edge2(src2, dst2, src, dst)

    ncls = b2.shape[0]
    out = pl.pallas_call(
        _fin_body,
        grid=(grid,),
        in_specs=[
            pl.BlockSpec((NC, nb, ACC2_W), lambda i: (0, i, 0)),
            pl.BlockSpec((1, ncls), lambda i: (0, 0)),
        ],
        out_specs=pl.BlockSpec((nb, ncls), lambda i: (i, 0)),
        out_shape=jax.ShapeDtypeStruct((n, ncls), jnp.float32),
    )(part2, b2.reshape(1, ncls))
    return out


# speedup vs baseline: 74.7861x; 1.0360x over previous
import functools

import jax
import jax.numpy as jnp
from jax import lax
from jax.experimental import pallas as pl
from jax.experimental.pallas import tpu as pltpu
from jax.experimental.pallas import tpu_sc as plsc

NC = 2
NS = 16
LN = 16

F_IN = 128
H = 8
C1 = 16
C2 = 40
SRC1_W = 144
DST1_W = 16
ACC1_W = 144
SRC2_W = 176
DST2_W = 16
ACC2_W = 168
K_E = 40
K_E2 = 32
ZR = 40


def _blockdiag(a):
    h, c = a.shape
    eye = jnp.eye(h, dtype=a.dtype)
    return (eye[:, None, :] * a[:, :, None]).reshape(h * c, h)


def _prep1_body(x_ref, w1_ref, asf_ref, adf_ref, src_ref, asad_ref):
    n = x_ref.shape[0]
    h1 = jnp.dot(x_ref[...], w1_ref[...], preferred_element_type=jnp.float32)
    as1 = jnp.dot(h1, asf_ref[...], preferred_element_type=jnp.float32)
    ad1 = jnp.dot(h1, adf_ref[...], preferred_element_type=jnp.float32)
    z8 = jnp.zeros((n, H), jnp.float32)
    src_ref[...] = jnp.concatenate([h1, as1, z8], axis=1)
    asad_ref[...] = jnp.concatenate([as1, ad1], axis=1)


def _dst1_body(asad_ref, dst_ref):
    asad = asad_ref[...]
    as1 = asad[:, 0:H]
    ad1 = asad[:, H:2 * H]
    maxas = jnp.max(as1, axis=0, keepdims=True)
    m1 = jnp.maximum(0.0, maxas + ad1)
    dst_ref[...] = jnp.concatenate([ad1, m1], axis=1)


def _zero_acc(acc, zbuf, sid, width):
    n = acc.shape[0]
    zrows = zbuf.shape[0]
    nchunks = n // zrows
    per_s = (nchunks + NS - 1) // NS
    nvec = (width + LN - 1) // LN

    def zero_row(r, _):
        for j in range(nvec):
            off = min(LN * j, width - LN)
            zbuf[r, pl.ds(off, LN)] = jnp.zeros((LN,), jnp.float32)
        return 0
    lax.fori_loop(0, zrows, zero_row, 0)
    for i in range(per_s):
        ci = sid + NS * i
        @pl.when(ci < nchunks)
        def _():
            pltpu.sync_copy(zbuf, acc.at[pl.ds(zrows * ci, zrows)])


def _write_acc(acc, out_plane, zbuf, sid):
    n = acc.shape[0]
    zrows = zbuf.shape[0]
    nchunks = n // zrows
    per_s = (nchunks + NS - 1) // NS
    for i in range(per_s):
        ci = sid + NS * i
        @pl.when(ci < nchunks)
        def _():
            sl = pl.ds(zrows * ci, zrows)
            pltpu.sync_copy(acc.at[sl], out_plane.at[sl])


def _edge1_body(src_t, dst_t, sidx, didx, out_hbm,
                acc, zbuf, msg,
                rs0, rs1, rd0, rd1, sv0, sv1, dv0, dv1,
                semg0, semg1, semi0, semi1):
    n = acc.shape[0]
    cid = lax.axis_index("c")
    sid = lax.axis_index("s")
    wid = cid * NS + sid
    e_per_w = sidx.shape[0] // (NC * NS)
    nt = e_per_w // K_E
    ebase0 = wid * e_per_w

    _zero_acc(acc, zbuf, sid, ACC1_W)
    plsc.subcore_barrier()

    lane = lax.iota(jnp.int32, LN)
    midx = jnp.minimum(lane + H, LN - 1)
    rs = [rs0, rs1]
    rd = [rd0, rd1]
    sv = [sv0, sv1]
    dv = [dv0, dv1]
    semg = [semg0, semg1]
    semi = [semi0, semi1]

    def issue_idx(b, t):
        base = ebase0 + t * K_E
        pltpu.async_copy(sidx.at[pl.ds(base, K_E)], sv[b], semi[b])
        pltpu.async_copy(didx.at[pl.ds(base, K_E)], dv[b], semi[b])

    def wait_idx(b):
        pltpu.make_async_copy(sidx.at[pl.ds(0, K_E)], sv[b], semi[b]).wait()
        pltpu.make_async_copy(didx.at[pl.ds(0, K_E)], dv[b], semi[b]).wait()

    def issue_gather(b):
        pltpu.async_copy(src_t.at[sv[b]], rs[b], semg[b])
        pltpu.async_copy(dst_t.at[dv[b]], rd[b], semg[b])

    def wait_gather(b):
        pltpu.make_async_copy(src_t.at[sv[b]], rs[b], semg[b]).wait()
        pltpu.make_async_copy(dst_t.at[dv[b]], rd[b], semg[b]).wait()

    def process(b):
        rows_s = rs[b]
        rows_d = rd[b]

        @plsc.parallel_loop(0, K_E, unroll=8)
        def edge_body(k):
            asv = rows_s[k, pl.ds(F_IN, LN)]
            adv = rows_d[k, pl.ds(0, LN)]
            mv = _lane_take(adv, midx)
            tv = asv + adv
            ev = jnp.maximum(tv, 0.2 * tv)
            w = jnp.exp(ev - mv)
            w = jnp.where(lane < H, w, 0.0)
            for hh in range(H):
                wh = _lane_bcast(w, hh)
                msg[k, pl.ds(LN * hh, LN)] = rows_s[k, pl.ds(LN * hh, LN)] * wh
            msg[k, pl.ds(F_IN, LN)] = w
        pltpu.sync_copy(msg, acc.at[dv[b]], add=True)

    issue_idx(0, 0)
    wait_idx(0)
    issue_gather(0)
    issue_idx(1, 1)

    def pair_body(u, _):
        t0 = 2 * u
        wait_gather(0)
        wait_idx(1)
        issue_gather(1)
        process(0)
        @pl.when(t0 + 2 < nt)
        def _():
            issue_idx(0, t0 + 2)
        wait_gather(1)
        @pl.when(t0 + 2 < nt)
        def _():
            wait_idx(0)
            issue_gather(0)
        process(1)
        @pl.when(t0 + 3 < nt)
        def _():
            issue_idx(1, t0 + 3)
        return 0
    lax.fori_loop(0, nt // 2, pair_body, 0)
    plsc.subcore_barrier()
    _write_acc(acc, out_hbm.at[cid], zbuf, sid)


_GDN = lax.GatherDimensionNumbers(
    offset_dims=(), collapsed_slice_dims=(0,), start_index_map=(0,))


def _lane_take(v, idx):
    return lax.gather(v, idx.reshape(LN, 1), _GDN, (1,),
                      mode=lax.GatherScatterMode.PROMISE_IN_BOUNDS)


def _lane_bcast(v, h):
    return _lane_take(v, lax.iota(jnp.int32, LN) * 0 + h)


def _lane_head(v, j):
    pos = lax.iota(jnp.int32, LN) + LN * j
    one = jnp.ones((LN,), jnp.int32)
    hidx = jnp.zeros((LN,), jnp.int32)
    for b in range(1, 4):
        hidx = hidx + jnp.where(pos >= b * C2, one, 0)
    return _lane_take(v, hidx)


def _mid_body(p_ref, b1_ref, w2_ref, asf_ref, adf_ref, eh_ref,
              src2_ref, asad_ref):
    p = p_ref[...]
    numer = p[0, :, 0:F_IN] + p[1, :, 0:F_IN]
    den = p[0, :, F_IN:F_IN + H] + p[1, :, F_IN:F_IN + H]
    recip = 1.0 / (den + 1e-16)
    rec128 = jnp.dot(recip, eh_ref[...], preferred_element_type=jnp.float32)
    o1 = numer * rec128 + b1_ref[...]
    o1 = jnp.where(o1 > 0, o1, jnp.exp(o1) - 1.0)
    h2 = jnp.dot(o1, w2_ref[...], preferred_element_type=jnp.float32)
    as2 = jnp.dot(h2, asf_ref[...], preferred_element_type=jnp.float32)
    ad2 = jnp.dot(h2, adf_ref[...], preferred_element_type=jnp.float32)
    bn = h2.shape[0]
    z12 = jnp.zeros((bn, 12), jnp.float32)
    halves = []
    for c in range(NC):
        halves.append(jnp.concatenate(
            [h2[:, 160 * c:160 * (c + 1)], as2[:, 4 * c:4 * (c + 1)], z12],
            axis=1))
    src2_ref[...] = jnp.stack(halves, axis=0)
    asad_ref[...] = jnp.concatenate([as2, ad2], axis=1)


def _dst2_body(asad_ref, dst2_ref):
    asad = asad_ref[...]
    as2 = asad[:, 0:H]
    ad2 = asad[:, H:2 * H]
    maxas = jnp.max(as2, axis=0, keepdims=True)
    m2 = jnp.maximum(0.0, maxas + ad2)
    n = asad.shape[0]
    z8 = jnp.zeros((n, H), jnp.float32)
    halves = []
    for c in range(NC):
        halves.append(jnp.concatenate(
            [ad2[:, 4 * c:4 * (c + 1)], m2[:, 4 * c:4 * (c + 1)], z8],
            axis=1))
    dst2_ref[...] = jnp.stack(halves, axis=0)


def _edge2_body(src_t, dst_t, sidx, didx, out_hbm,
                acc, zbuf, msg,
                rs0, rs1, rd0, rd1, sv0, sv1, dv0, dv1,
                semg0, semg1, semi0, semi1):
    n = acc.shape[0]
    cid = lax.axis_index("c")
    sid = lax.axis_index("s")
    e_per_s = sidx.shape[0] // NS
    nt = e_per_s // K_E2
    ebase0 = sid * e_per_s

    _zero_acc(acc, zbuf, sid, ACC2_W)
    plsc.subcore_barrier()

    lane = lax.iota(jnp.int32, LN)
    midx = jnp.minimum(lane + 4, LN - 1)
    hiidx = jnp.minimum(lane + H, LN - 1)
    loidx = jnp.maximum(lane - H, 0)
    rs = [rs0, rs1]
    rd = [rd0, rd1]
    sv = [sv0, sv1]
    dv = [dv0, dv1]
    semg = [semg0, semg1]
    semi = [semi0, semi1]

    def issue_idx(b, t):
        base = ebase0 + t * K_E2
        pltpu.async_copy(sidx.at[pl.ds(base, K_E2)], sv[b], semi[b])
        pltpu.async_copy(didx.at[pl.ds(base, K_E2)], dv[b], semi[b])

    def wait_idx(b):
        pltpu.make_async_copy(sidx.at[pl.ds(0, K_E2)], sv[b], semi[b]).wait()
        pltpu.make_async_copy(didx.at[pl.ds(0, K_E2)], dv[b], semi[b]).wait()

    def issue_gather(b):
        pltpu.async_copy(src_t.at[cid].at[sv[b]], rs[b], semg[b])
        pltpu.async_copy(dst_t.at[cid].at[dv[b]], rd[b], semg[b])

    def wait_gather(b):
        pltpu.make_async_copy(src_t.at[cid].at[sv[b]], rs[b], semg[b]).wait()
        pltpu.make_async_copy(dst_t.at[cid].at[dv[b]], rd[b], semg[b]).wait()

    def process(b):
        rows_s = rs[b]
        rows_d = rd[b]

        @plsc.parallel_loop(0, K_E2, unroll=8)
        def edge_body(k):
            asv = rows_s[k, pl.ds(160, LN)]
            adv = rows_d[k, pl.ds(0, LN)]
            mv = _lane_take(adv, midx)
            tv = asv + adv
            ev = jnp.maximum(tv, 0.2 * tv)
            w = jnp.exp(ev - mv)
            w = jnp.where(lane < 4, w, 0.0)
            vj9 = None
            for j in range(160 // LN):
                wl = _lane_head(w, j)
                vj = rows_s[k, pl.ds(LN * j, LN)] * wl
                msg[k, pl.ds(LN * j, LN)] = vj
                vj9 = vj
            comb = jnp.where(lane < H, _lane_take(vj9, hiidx),
                             _lane_take(w, loidx))
            msg[k, pl.ds(152, LN)] = comb
        pltpu.sync_copy(msg, acc.at[dv[b]], add=True)

    issue_idx(0, 0)
    wait_idx(0)
    issue_gather(0)
    issue_idx(1, 1)

    def pair_body(u, _):
        t0 = 2 * u
        wait_gather(0)
        wait_idx(1)
        issue_gather(1)
        process(0)
        @pl.when(t0 + 2 < nt)
        def _():
            issue_idx(0, t0 + 2)
        wait_gather(1)
        @pl.when(t0 + 2 < nt)
        def _():
            wait_idx(0)
            issue_gather(0)
        process(1)
        @pl.when(t0 + 3 < nt)
        def _():
            issue_idx(1, t0 + 3)
        return 0
    lax.fori_loop(0, nt // 2, pair_body, 0)
    if nt % 2 == 1:
        wait_gather(0)
        process(0)
    plsc.subcore_barrier()
    _write_acc(acc, out_hbm.at[cid], zbuf, sid)


def _fin_body(a_ref, b2_ref, o_ref):
    a = a_ref[...]
    s = jnp.zeros(o_ref.shape, jnp.float32)
    for g in range(NC):
        for j in range(4):
            numer = a[g, :, C2 * j:C2 * (j + 1)]
            den = a[g, :, 160 + j][:, None]
            s = s + numer / (den + 1e-16)
    o = s * (1.0 / H) + b2_ref[...]
    m = jnp.max(o, axis=1, keepdims=True)
    z = o - m
    lse = jnp.log(jnp.sum(jnp.exp(z), axis=1, keepdims=True))
    o_ref[...] = z - lse


def kernel(x, edge_index, W1, a_s1, a_d1, b1, W2, a_s2, a_d2, b2):
    n = x.shape[0]
    e = edge_index.shape[1]
    src = edge_index[0].astype(jnp.int32)
    dst = edge_index[1].astype(jnp.int32)

    asf1 = _blockdiag(a_s1)
    adf1 = _blockdiag(a_d1)
    asf2 = _blockdiag(a_s2)
    adf2 = _blockdiag(a_d2)
    eh = jnp.repeat(jnp.eye(H, dtype=jnp.float32), C1, axis=1)

    nb = 2000
    grid = n // nb
    src1, asad1 = pl.pallas_call(
        _prep1_body,
        grid=(grid,),
        in_specs=[
            pl.BlockSpec((nb, F_IN), lambda i: (i, 0)),
            pl.BlockSpec((F_IN, H * C1), lambda i: (0, 0)),
            pl.BlockSpec((H * C1, H), lambda i: (0, 0)),
            pl.BlockSpec((H * C1, H), lambda i: (0, 0)),
        ],
        out_specs=[
            pl.BlockSpec((nb, SRC1_W), lambda i: (i, 0)),
            pl.BlockSpec((nb, 2 * H), lambda i: (i, 0)),
        ],
        out_shape=(jax.ShapeDtypeStruct((n, SRC1_W), jnp.float32),
                   jax.ShapeDtypeStruct((n, 2 * H), jnp.float32)),
    )(x, W1, asf1, adf1)

    dst1 = pl.pallas_call(
        _dst1_body,
        out_shape=jax.ShapeDtypeStruct((n, DST1_W), jnp.float32),
    )(asad1)

    edge1 = pl.kernel(
        _edge1_body,
        out_type=jax.ShapeDtypeStruct((NC, n, ACC1_W), jnp.float32),
        mesh=plsc.VectorSubcoreMesh(core_axis_name="c", subcore_axis_name="s"),
        compiler_params=pltpu.CompilerParams(use_tc_tiling_on_sc=False),
        scratch_types=[
            pltpu.VMEM_SHARED((n, ACC1_W), jnp.float32),
            pltpu.VMEM((ZR, ACC1_W), jnp.float32),
            pltpu.VMEM((K_E, ACC1_W), jnp.float32),
            pltpu.VMEM((K_E, SRC1_W), jnp.float32),
            pltpu.VMEM((K_E, SRC1_W), jnp.float32),
            pltpu.VMEM((K_E, DST1_W), jnp.float32),
            pltpu.VMEM((K_E, DST1_W), jnp.float32),
            pltpu.VMEM((K_E,), jnp.int32),
            pltpu.VMEM((K_E,), jnp.int32),
            pltpu.VMEM((K_E,), jnp.int32),
            pltpu.VMEM((K_E,), jnp.int32),
            pltpu.SemaphoreType.DMA,
            pltpu.SemaphoreType.DMA,
            pltpu.SemaphoreType.DMA,
            pltpu.SemaphoreType.DMA,
        ],
    )
    part1 = edge1(src1, dst1, src, dst)

    src2, asad = pl.pallas_call(
        _mid_body,
        grid=(grid,),
        in_specs=[
            pl.BlockSpec((NC, nb, ACC1_W), lambda i: (0, i, 0)),
            pl.BlockSpec((1, F_IN), lambda i: (0, 0)),
            pl.BlockSpec((F_IN, H * C2), lambda i: (0, 0)),
            pl.BlockSpec((H * C2, H), lambda i: (0, 0)),
            pl.BlockSpec((H * C2, H), lambda i: (0, 0)),
            pl.BlockSpec((H, F_IN), lambda i: (0, 0)),
        ],
        out_specs=[
            pl.BlockSpec((NC, nb, SRC2_W), lambda i: (0, i, 0)),
            pl.BlockSpec((nb, 2 * H), lambda i: (i, 0)),
        ],
        out_shape=(jax.ShapeDtypeStruct((NC, n, SRC2_W), jnp.float32),
                   jax.ShapeDtypeStruct((n, 2 * H), jnp.float32)),
    )(part1, b1.reshape(1, F_IN), W2, asf2, adf2, eh)

    dst2 = pl.pallas_call(
        _dst2_body,
        out_shape=jax.ShapeDtypeStruct((NC, n, DST2_W), jnp.float32),
    )(asad)

    edge2 = pl.kernel(
        _edge2_body,
        out_type=jax.ShapeDtypeStruct((NC, n, ACC2_W), jnp.float32),
        mesh=plsc.VectorSubcoreMesh(core_axis_name="c", subcore_axis_name="s"),
        compiler_params=pltpu.CompilerParams(use_tc_tiling_on_sc=False),
        scratch_types=[
            pltpu.VMEM_SHARED((n, ACC2_W), jnp.float32),
            pltpu.VMEM((ZR, ACC2_W), jnp.float32),
            pltpu.VMEM((K_E2, ACC2_W), jnp.float32),
            pltpu.VMEM((K_E2, SRC2_W), jnp.float32),
            pltpu.VMEM((K_E2, SRC2_W), jnp.float32),
            pltpu.VMEM((K_E2, DST2_W), jnp.float32),
            pltpu.VMEM((K_E2, DST2_W), jnp.float32),
            pltpu.VMEM((K_E2,), jnp.int32),
            pltpu.VMEM((K_E2,), jnp.int32),
            pltpu.VMEM((K_E2,), jnp.int32),
            pltpu.VMEM((K_E2,), jnp.int32),
            pltpu.SemaphoreType.DMA,
            pltpu.SemaphoreType.DMA,
            pltpu.SemaphoreType.DMA,
            pltpu.SemaphoreType.DMA,
        ],
    )
    part2 = edge2(src2, dst2, src, dst)

    ncls = b2.shape[0]
    out = pl.pallas_call(
        _fin_body,
        grid=(grid,),
        in_specs=[
            pl.BlockSpec((NC, nb, ACC2_W), lambda i: (0, i, 0)),
            pl.BlockSpec((1, ncls), lambda i: (0, 0)),
        ],
        out_specs=pl.BlockSpec((nb, ncls), lambda i: (i, 0)),
        out_shape=jax.ShapeDtypeStruct((n, ncls), jnp.float32),
    )(part2, b2.reshape(1, ncls))
    return out


# speedup vs baseline: 78.6596x; 1.0518x over previous
import functools

import jax
import jax.numpy as jnp
from jax import lax
from jax.experimental import pallas as pl
from jax.experimental.pallas import tpu as pltpu
from jax.experimental.pallas import tpu_sc as plsc

NC = 2
NS = 16
LN = 16

F_IN = 128
H = 8
C1 = 16
C2 = 40
SRC1_W = 144
DST1_W = 16
ACC1_W = 144
SRC2_W = 176
DST2_W = 16
ACC2_W = 168
K_E = 40
K_E2 = 40
ZR = 40
ZR2 = 20


def _blockdiag(a):
    h, c = a.shape
    eye = jnp.eye(h, dtype=a.dtype)
    return (eye[:, None, :] * a[:, :, None]).reshape(h * c, h)


def _prep1_body(x_ref, w1_ref, asf_ref, adf_ref, src_ref, asad_ref):
    n = x_ref.shape[0]
    h1 = jnp.dot(x_ref[...], w1_ref[...], preferred_element_type=jnp.float32)
    as1 = jnp.dot(h1, asf_ref[...], preferred_element_type=jnp.float32)
    ad1 = jnp.dot(h1, adf_ref[...], preferred_element_type=jnp.float32)
    z8 = jnp.zeros((n, H), jnp.float32)
    src_ref[...] = jnp.concatenate([h1, as1, z8], axis=1)
    asad_ref[...] = jnp.concatenate([as1, ad1], axis=1)


def _dst1_body(asad_ref, dst_ref):
    asad = asad_ref[...]
    as1 = asad[:, 0:H]
    ad1 = asad[:, H:2 * H]
    maxas = jnp.max(as1, axis=0, keepdims=True)
    m1 = jnp.maximum(0.0, maxas + ad1)
    dst_ref[...] = jnp.concatenate([ad1, m1], axis=1)


def _zero_acc(acc, zbuf, sid, width):
    n = acc.shape[0]
    zrows = zbuf.shape[0]
    nchunks = n // zrows
    per_s = (nchunks + NS - 1) // NS
    nvec = (width + LN - 1) // LN

    def zero_row(r, _):
        for j in range(nvec):
            off = min(LN * j, width - LN)
            zbuf[r, pl.ds(off, LN)] = jnp.zeros((LN,), jnp.float32)
        return 0
    lax.fori_loop(0, zrows, zero_row, 0)
    for i in range(per_s):
        ci = sid + NS * i
        @pl.when(ci < nchunks)
        def _():
            pltpu.sync_copy(zbuf, acc.at[pl.ds(zrows * ci, zrows)])


def _write_acc(acc, out_plane, zbuf, sid):
    n = acc.shape[0]
    zrows = zbuf.shape[0]
    nchunks = n // zrows
    per_s = (nchunks + NS - 1) // NS
    for i in range(per_s):
        ci = sid + NS * i
        @pl.when(ci < nchunks)
        def _():
            sl = pl.ds(zrows * ci, zrows)
            pltpu.sync_copy(acc.at[sl], out_plane.at[sl])


def _edge1_body(src_t, dst_t, sidx, didx, out_hbm,
                acc, zbuf, msg,
                rs0, rs1, rd0, rd1, sv0, sv1, dv0, dv1,
                semg0, semg1, semi0, semi1):
    n = acc.shape[0]
    cid = lax.axis_index("c")
    sid = lax.axis_index("s")
    wid = cid * NS + sid
    e_per_w = sidx.shape[0] // (NC * NS)
    nt = e_per_w // K_E
    ebase0 = wid * e_per_w

    _zero_acc(acc, zbuf, sid, ACC1_W)
    plsc.subcore_barrier()

    lane = lax.iota(jnp.int32, LN)
    midx = jnp.minimum(lane + H, LN - 1)
    rs = [rs0, rs1]
    rd = [rd0, rd1]
    sv = [sv0, sv1]
    dv = [dv0, dv1]
    semg = [semg0, semg1]
    semi = [semi0, semi1]

    def issue_idx(b, t):
        base = ebase0 + t * K_E
        pltpu.async_copy(sidx.at[pl.ds(base, K_E)], sv[b], semi[b])
        pltpu.async_copy(didx.at[pl.ds(base, K_E)], dv[b], semi[b])

    def wait_idx(b):
        pltpu.make_async_copy(sidx.at[pl.ds(0, K_E)], sv[b], semi[b]).wait()
        pltpu.make_async_copy(didx.at[pl.ds(0, K_E)], dv[b], semi[b]).wait()

    def issue_gather(b):
        pltpu.async_copy(src_t.at[sv[b]], rs[b], semg[b])
        pltpu.async_copy(dst_t.at[dv[b]], rd[b], semg[b])

    def wait_gather(b):
        pltpu.make_async_copy(src_t.at[sv[b]], rs[b], semg[b]).wait()
        pltpu.make_async_copy(dst_t.at[dv[b]], rd[b], semg[b]).wait()

    def process(b):
        rows_s = rs[b]
        rows_d = rd[b]

        @plsc.parallel_loop(0, K_E, unroll=8)
        def edge_body(k):
            asv = rows_s[k, pl.ds(F_IN, LN)]
            adv = rows_d[k, pl.ds(0, LN)]
            mv = _lane_take(adv, midx)
            tv = asv + adv
            ev = jnp.maximum(tv, 0.2 * tv)
            w = jnp.exp(ev - mv)
            w = jnp.where(lane < H, w, 0.0)
            for hh in range(H):
                wh = _lane_bcast(w, hh)
                msg[k, pl.ds(LN * hh, LN)] = rows_s[k, pl.ds(LN * hh, LN)] * wh
            msg[k, pl.ds(F_IN, LN)] = w
        pltpu.sync_copy(msg, acc.at[dv[b]], add=True)

    issue_idx(0, 0)
    wait_idx(0)
    issue_gather(0)
    issue_idx(1, 1)

    def pair_body(u, _):
        t0 = 2 * u
        wait_gather(0)
        wait_idx(1)
        issue_gather(1)
        process(0)
        @pl.when(t0 + 2 < nt)
        def _():
            issue_idx(0, t0 + 2)
        wait_gather(1)
        @pl.when(t0 + 2 < nt)
        def _():
            wait_idx(0)
            issue_gather(0)
        process(1)
        @pl.when(t0 + 3 < nt)
        def _():
            issue_idx(1, t0 + 3)
        return 0
    lax.fori_loop(0, nt // 2, pair_body, 0)
    plsc.subcore_barrier()
    _write_acc(acc, out_hbm.at[cid], zbuf, sid)


_GDN = lax.GatherDimensionNumbers(
    offset_dims=(), collapsed_slice_dims=(0,), start_index_map=(0,))


def _lane_take(v, idx):
    return lax.gather(v, idx.reshape(LN, 1), _GDN, (1,),
                      mode=lax.GatherScatterMode.PROMISE_IN_BOUNDS)


def _lane_bcast(v, h):
    return _lane_take(v, lax.iota(jnp.int32, LN) * 0 + h)


def _lane_head(v, j):
    pos = lax.iota(jnp.int32, LN) + LN * j
    one = jnp.ones((LN,), jnp.int32)
    hidx = jnp.zeros((LN,), jnp.int32)
    for b in range(1, 4):
        hidx = hidx + jnp.where(pos >= b * C2, one, 0)
    return _lane_take(v, hidx)


def _mid_body(p_ref, b1_ref, w2_ref, asf_ref, adf_ref, eh_ref,
              src2_ref, asad_ref):
    p = p_ref[...]
    numer = p[0, :, 0:F_IN] + p[1, :, 0:F_IN]
    den = p[0, :, F_IN:F_IN + H] + p[1, :, F_IN:F_IN + H]
    recip = 1.0 / (den + 1e-16)
    rec128 = jnp.dot(recip, eh_ref[...], preferred_element_type=jnp.float32)
    o1 = numer * rec128 + b1_ref[...]
    o1 = jnp.where(o1 > 0, o1, jnp.exp(o1) - 1.0)
    h2 = jnp.dot(o1, w2_ref[...], preferred_element_type=jnp.float32)
    as2 = jnp.dot(h2, asf_ref[...], preferred_element_type=jnp.float32)
    ad2 = jnp.dot(h2, adf_ref[...], preferred_element_type=jnp.float32)
    bn = h2.shape[0]
    z12 = jnp.zeros((bn, 12), jnp.float32)
    halves = []
    for c in range(NC):
        halves.append(jnp.concatenate(
            [h2[:, 160 * c:160 * (c + 1)], as2[:, 4 * c:4 * (c + 1)], z12],
            axis=1))
    src2_ref[...] = jnp.stack(halves, axis=0)
    asad_ref[...] = jnp.concatenate([as2, ad2], axis=1)


def _dst2_body(asad_ref, dst2_ref):
    asad = asad_ref[...]
    as2 = asad[:, 0:H]
    ad2 = asad[:, H:2 * H]
    maxas = jnp.max(as2, axis=0, keepdims=True)
    m2 = jnp.maximum(0.0, maxas + ad2)
    n = asad.shape[0]
    z8 = jnp.zeros((n, H), jnp.float32)
    halves = []
    for c in range(NC):
        halves.append(jnp.concatenate(
            [ad2[:, 4 * c:4 * (c + 1)], m2[:, 4 * c:4 * (c + 1)], z8],
            axis=1))
    dst2_ref[...] = jnp.stack(halves, axis=0)


def _edge2_body(src_t, dst_t, sidx, didx, out_hbm,
                acc, zbuf, msg,
                rs0, rs1, rd0, rd1, sv0, sv1, dv0, dv1,
                semg0, semg1, semi0, semi1):
    n = acc.shape[0]
    cid = lax.axis_index("c")
    sid = lax.axis_index("s")
    e_per_s = sidx.shape[0] // NS
    nt = e_per_s // K_E2
    ebase0 = sid * e_per_s

    _zero_acc(acc, zbuf, sid, ACC2_W)
    plsc.subcore_barrier()

    lane = lax.iota(jnp.int32, LN)
    midx = jnp.minimum(lane + 4, LN - 1)
    hiidx = jnp.minimum(lane + H, LN - 1)
    loidx = jnp.maximum(lane - H, 0)
    rs = [rs0, rs1]
    rd = [rd0, rd1]
    sv = [sv0, sv1]
    dv = [dv0, dv1]
    semg = [semg0, semg1]
    semi = [semi0, semi1]

    def issue_idx(b, t):
        base = ebase0 + t * K_E2
        pltpu.async_copy(sidx.at[pl.ds(base, K_E2)], sv[b], semi[b])
        pltpu.async_copy(didx.at[pl.ds(base, K_E2)], dv[b], semi[b])

    def wait_idx(b):
        pltpu.make_async_copy(sidx.at[pl.ds(0, K_E2)], sv[b], semi[b]).wait()
        pltpu.make_async_copy(didx.at[pl.ds(0, K_E2)], dv[b], semi[b]).wait()

    def issue_gather(b):
        pltpu.async_copy(src_t.at[cid].at[sv[b]], rs[b], semg[b])
        pltpu.async_copy(dst_t.at[cid].at[dv[b]], rd[b], semg[b])

    def wait_gather(b):
        pltpu.make_async_copy(src_t.at[cid].at[sv[b]], rs[b], semg[b]).wait()
        pltpu.make_async_copy(dst_t.at[cid].at[dv[b]], rd[b], semg[b]).wait()

    def process(b):
        rows_s = rs[b]
        rows_d = rd[b]

        @plsc.parallel_loop(0, K_E2, unroll=8)
        def edge_body(k):
            asv = rows_s[k, pl.ds(160, LN)]
            adv = rows_d[k, pl.ds(0, LN)]
            mv = _lane_take(adv, midx)
            tv = asv + adv
            ev = jnp.maximum(tv, 0.2 * tv)
            w = jnp.exp(ev - mv)
            w = jnp.where(lane < 4, w, 0.0)
            vj9 = None
            for j in range(160 // LN):
                wl = _lane_head(w, j)
                vj = rows_s[k, pl.ds(LN * j, LN)] * wl
                msg[k, pl.ds(LN * j, LN)] = vj
                vj9 = vj
            comb = jnp.where(lane < H, _lane_take(vj9, hiidx),
                             _lane_take(w, loidx))
            msg[k, pl.ds(152, LN)] = comb
        pltpu.sync_copy(msg, acc.at[dv[b]], add=True)

    issue_idx(0, 0)
    wait_idx(0)
    issue_gather(0)
    issue_idx(1, 1)

    def pair_body(u, _):
        t0 = 2 * u
        wait_gather(0)
        wait_idx(1)
        issue_gather(1)
        process(0)
        @pl.when(t0 + 2 < nt)
        def _():
            issue_idx(0, t0 + 2)
        wait_gather(1)
        @pl.when(t0 + 2 < nt)
        def _():
            wait_idx(0)
            issue_gather(0)
        process(1)
        @pl.when(t0 + 3 < nt)
        def _():
            issue_idx(1, t0 + 3)
        return 0
    lax.fori_loop(0, nt // 2, pair_body, 0)
    if nt % 2 == 1:
        wait_gather(0)
        process(0)
    plsc.subcore_barrier()
    _write_acc(acc, out_hbm.at[cid], zbuf, sid)


def _fin_body(a_ref, b2_ref, o_ref):
    a = a_ref[...]
    s = jnp.zeros(o_ref.shape, jnp.float32)
    for g in range(NC):
        for j in range(4):
            numer = a[g, :, C2 * j:C2 * (j + 1)]
            den = a[g, :, 160 + j][:, None]
            s = s + numer / (den + 1e-16)
    o = s * (1.0 / H) + b2_ref[...]
    m = jnp.max(o, axis=1, keepdims=True)
    z = o - m
    lse = jnp.log(jnp.sum(jnp.exp(z), axis=1, keepdims=True))
    o_ref[...] = z - lse


def kernel(x, edge_index, W1, a_s1, a_d1, b1, W2, a_s2, a_d2, b2):
    n = x.shape[0]
    e = edge_index.shape[1]
    src = edge_index[0].astype(jnp.int32)
    dst = edge_index[1].astype(jnp.int32)

    asf1 = _blockdiag(a_s1)
    adf1 = _blockdiag(a_d1)
    asf2 = _blockdiag(a_s2)
    adf2 = _blockdiag(a_d2)
    eh = jnp.repeat(jnp.eye(H, dtype=jnp.float32), C1, axis=1)

    nb = 2000
    grid = n // nb
    src1, asad1 = pl.pallas_call(
        _prep1_body,
        grid=(grid,),
        in_specs=[
            pl.BlockSpec((nb, F_IN), lambda i: (i, 0)),
            pl.BlockSpec((F_IN, H * C1), lambda i: (0, 0)),
            pl.BlockSpec((H * C1, H), lambda i: (0, 0)),
            pl.BlockSpec((H * C1, H), lambda i: (0, 0)),
        ],
        out_specs=[
            pl.BlockSpec((nb, SRC1_W), lambda i: (i, 0)),
            pl.BlockSpec((nb, 2 * H), lambda i: (i, 0)),
        ],
        out_shape=(jax.ShapeDtypeStruct((n, SRC1_W), jnp.float32),
                   jax.ShapeDtypeStruct((n, 2 * H), jnp.float32)),
    )(x, W1, asf1, adf1)

    dst1 = pl.pallas_call(
        _dst1_body,
        out_shape=jax.ShapeDtypeStruct((n, DST1_W), jnp.float32),
    )(asad1)

    edge1 = pl.kernel(
        _edge1_body,
        out_type=jax.ShapeDtypeStruct((NC, n, ACC1_W), jnp.float32),
        mesh=plsc.VectorSubcoreMesh(core_axis_name="c", subcore_axis_name="s"),
        compiler_params=pltpu.CompilerParams(use_tc_tiling_on_sc=False),
        scratch_types=[
            pltpu.VMEM_SHARED((n, ACC1_W), jnp.float32),
            pltpu.VMEM((ZR, ACC1_W), jnp.float32),
            pltpu.VMEM((K_E, ACC1_W), jnp.float32),
            pltpu.VMEM((K_E, SRC1_W), jnp.float32),
            pltpu.VMEM((K_E, SRC1_W), jnp.float32),
            pltpu.VMEM((K_E, DST1_W), jnp.float32),
            pltpu.VMEM((K_E, DST1_W), jnp.float32),
            pltpu.VMEM((K_E,), jnp.int32),
            pltpu.VMEM((K_E,), jnp.int32),
            pltpu.VMEM((K_E,), jnp.int32),
            pltpu.VMEM((K_E,), jnp.int32),
            pltpu.SemaphoreType.DMA,
            pltpu.SemaphoreType.DMA,
            pltpu.SemaphoreType.DMA,
            pltpu.SemaphoreType.DMA,
        ],
    )
    part1 = edge1(src1, dst1, src, dst)

    src2, asad = pl.pallas_call(
        _mid_body,
        grid=(grid,),
        in_specs=[
            pl.BlockSpec((NC, nb, ACC1_W), lambda i: (0, i, 0)),
            pl.BlockSpec((1, F_IN), lambda i: (0, 0)),
            pl.BlockSpec((F_IN, H * C2), lambda i: (0, 0)),
            pl.BlockSpec((H * C2, H), lambda i: (0, 0)),
            pl.BlockSpec((H * C2, H), lambda i: (0, 0)),
            pl.BlockSpec((H, F_IN), lambda i: (0, 0)),
        ],
        out_specs=[
            pl.BlockSpec((NC, nb, SRC2_W), lambda i: (0, i, 0)),
            pl.BlockSpec((nb, 2 * H), lambda i: (i, 0)),
        ],
        out_shape=(jax.ShapeDtypeStruct((NC, n, SRC2_W), jnp.float32),
                   jax.ShapeDtypeStruct((n, 2 * H), jnp.float32)),
    )(part1, b1.reshape(1, F_IN), W2, asf2, adf2, eh)

    dst2 = pl.pallas_call(
        _dst2_body,
        out_shape=jax.ShapeDtypeStruct((NC, n, DST2_W), jnp.float32),
    )(asad)

    edge2 = pl.kernel(
        _edge2_body,
        out_type=jax.ShapeDtypeStruct((NC, n, ACC2_W), jnp.float32),
        mesh=plsc.VectorSubcoreMesh(core_axis_name="c", subcore_axis_name="s"),
        compiler_params=pltpu.CompilerParams(use_tc_tiling_on_sc=False),
        scratch_types=[
            pltpu.VMEM_SHARED((n, ACC2_W), jnp.float32),
            pltpu.VMEM((ZR2, ACC2_W), jnp.float32),
            pltpu.VMEM((K_E2, ACC2_W), jnp.float32),
            pltpu.VMEM((K_E2, SRC2_W), jnp.float32),
            pltpu.VMEM((K_E2, SRC2_W), jnp.float32),
            pltpu.VMEM((K_E2, DST2_W), jnp.float32),
            pltpu.VMEM((K_E2, DST2_W), jnp.float32),
            pltpu.VMEM((K_E2,), jnp.int32),
            pltpu.VMEM((K_E2,), jnp.int32),
            pltpu.VMEM((K_E2,), jnp.int32),
            pltpu.VMEM((K_E2,), jnp.int32),
            pltpu.SemaphoreType.DMA,
            pltpu.SemaphoreType.DMA,
            pltpu.SemaphoreType.DMA,
            pltpu.SemaphoreType.DMA,
        ],
    )
    part2 = edge2(src2, dst2, src, dst)

    ncls = b2.shape[0]
    out = pl.pallas_call(
        _fin_body,
        grid=(grid,),
        in_specs=[
            pl.BlockSpec((NC, nb, ACC2_W), lambda i: (0, i, 0)),
            pl.BlockSpec((1, ncls), lambda i: (0, 0)),
        ],
        out_specs=pl.BlockSpec((nb, ncls), lambda i: (i, 0)),
        out_shape=jax.ShapeDtypeStruct((n, ncls), jnp.float32),
    )(part2, b2.reshape(1, ncls))
    return out
